# baseline (speedup 1.0000x reference)
import jax
import jax.numpy as jnp
from jax.experimental import pallas as pl

B = 1024
NUM_CLASSES = 100000
D = 512

SCALE = 32.0
L_A = 10.0
U_A = 110.0
LAMDA = 20.0

_FLAT_ROWS = 6400
_FLAT_COLS = 16000
_BLK_ROWS = 128


def _scale_kernel(x_ref, o_ref):
    o_ref[...] = jnp.clip(x_ref[...], -1.0, 1.0) * SCALE


def _loss_g_kernel(f_ref, o_ref):
    f = f_ref[...]
    sq = jnp.sum(f * f, axis=1, keepdims=True)
    x_norm = jnp.clip(jnp.sqrt(sq), L_A, U_A)
    o_ref[...] = LAMDA * ((1.0 / (U_A * U_A)) * x_norm + 1.0 / x_norm)


def kernel(cos_theta, feats, labels):
    flat = cos_theta.reshape(_FLAT_ROWS, _FLAT_COLS)
    out = pl.pallas_call(
        _scale_kernel,
        out_shape=jax.ShapeDtypeStruct((_FLAT_ROWS, _FLAT_COLS), jnp.float32),
        grid=(_FLAT_ROWS // _BLK_ROWS,),
        in_specs=[pl.BlockSpec((_BLK_ROWS, _FLAT_COLS), lambda i: (i, 0))],
        out_specs=pl.BlockSpec((_BLK_ROWS, _FLAT_COLS), lambda i: (i, 0)),
    )(flat)
    loss_g = pl.pallas_call(
        _loss_g_kernel,
        out_shape=jax.ShapeDtypeStruct((B, 1), jnp.float32),
        in_specs=[pl.BlockSpec((B, D), lambda: (0, 0))],
        out_specs=pl.BlockSpec((B, 1), lambda: (0, 0)),
    )(feats)
    return (out.reshape(B, NUM_CLASSES), loss_g)


# speedup vs baseline: 2.1635x; 2.1635x over previous
import jax
import jax.numpy as jnp
from jax.experimental import pallas as pl

B = 1024
NUM_CLASSES = 100000
D = 512

SCALE = 32.0
L_A = 10.0
U_A = 110.0
LAMDA = 20.0

_BLK_ROWS = 32


def _scale_kernel(x_ref, o_ref):
    o_ref[...] = jnp.clip(x_ref[...], -1.0, 1.0) * SCALE


def _loss_g_kernel(f_ref, o_ref):
    f = f_ref[...]
    sq = jnp.sum(f * f, axis=1, keepdims=True)
    x_norm = jnp.clip(jnp.sqrt(sq), L_A, U_A)
    o_ref[...] = LAMDA * ((1.0 / (U_A * U_A)) * x_norm + 1.0 / x_norm)


def kernel(cos_theta, feats, labels):
    out = pl.pallas_call(
        _scale_kernel,
        out_shape=jax.ShapeDtypeStruct((B, NUM_CLASSES), jnp.float32),
        grid=(B // _BLK_ROWS,),
        in_specs=[pl.BlockSpec((_BLK_ROWS, NUM_CLASSES), lambda i: (i, 0))],
        out_specs=pl.BlockSpec((_BLK_ROWS, NUM_CLASSES), lambda i: (i, 0)),
    )(cos_theta)
    loss_g = pl.pallas_call(
        _loss_g_kernel,
        out_shape=jax.ShapeDtypeStruct((B, 1), jnp.float32),
        in_specs=[pl.BlockSpec((B, D), lambda: (0, 0))],
        out_specs=pl.BlockSpec((B, 1), lambda: (0, 0)),
    )(feats)
    return (out, loss_g)


# speedup vs baseline: 2.1683x; 1.0022x over previous
import jax
import jax.numpy as jnp
from jax.experimental import pallas as pl

B = 1024
NUM_CLASSES = 100000
D = 512

SCALE = 32.0
L_A = 10.0
U_A = 110.0
LAMDA = 20.0

_BLK_ROWS = 256
_BLK_COLS = 12800


def _scale_kernel(x_ref, o_ref):
    o_ref[...] = jnp.clip(x_ref[...], -1.0, 1.0) * SCALE


def _loss_g_kernel(f_ref, o_ref):
    f = f_ref[...]
    sq = jnp.sum(f * f, axis=1, keepdims=True)
    x_norm = jnp.clip(jnp.sqrt(sq), L_A, U_A)
    o_ref[...] = LAMDA * ((1.0 / (U_A * U_A)) * x_norm + 1.0 / x_norm)


def kernel(cos_theta, feats, labels):
    out = pl.pallas_call(
        _scale_kernel,
        out_shape=jax.ShapeDtypeStruct((B, NUM_CLASSES), jnp.float32),
        grid=(B // _BLK_ROWS, pl.cdiv(NUM_CLASSES, _BLK_COLS)),
        in_specs=[pl.BlockSpec((_BLK_ROWS, _BLK_COLS), lambda i, j: (i, j))],
        out_specs=pl.BlockSpec((_BLK_ROWS, _BLK_COLS), lambda i, j: (i, j)),
    )(cos_theta)
    loss_g = pl.pallas_call(
        _loss_g_kernel,
        out_shape=jax.ShapeDtypeStruct((B, 1), jnp.float32),
        in_specs=[pl.BlockSpec((B, D), lambda: (0, 0))],
        out_specs=pl.BlockSpec((B, 1), lambda: (0, 0)),
    )(feats)
    return (out, loss_g)
